# Initial kernel scaffold; baseline (speedup 1.0000x reference)
import jax
import jax.numpy as jnp
from jax.experimental import pallas as pl


def kernel(sample_points, points_frames, K):
    raise NotImplementedError("write your pallas kernel here")


# speedup vs baseline: 568.2664x; 568.2664x over previous
import jax
import jax.numpy as jnp
from jax.experimental import pallas as pl

_K = 16
_NB = 64


def _knn_body(q_ref, p_ref, o_ref):
    nb = q_ref.shape[1]
    fp = p_ref.shape[2]
    p = p_ref.shape[2] // 4

    q = q_ref[0]
    pts = p_ref[0]
    qc = q[:, :3]
    qf = q[:, 4].astype(jnp.int32) - 1

    col = jax.lax.broadcasted_iota(jnp.int32, (nb, fp), 1)

    dx = qc[:, 0:1] - pts[0:1, :]
    d2 = dx * dx
    dy = qc[:, 1:2] - pts[1:2, :]
    d2 = d2 + dy * dy
    dz = qc[:, 2:3] - pts[2:3, :]
    d2 = d2 + dz * dz

    other = (col // p) != qf[:, None]
    inf = jnp.float32(jnp.inf)
    d2 = jnp.where(other, inf, d2)

    feats = []
    for _ in range(_K):
        m = jnp.min(d2, axis=1)
        sel = d2 <= m[:, None]
        amin = jnp.min(jnp.where(sel, col, fp), axis=1)
        hit = col == amin[:, None]
        d2 = jnp.where(hit, inf, d2)
        oh = hit.astype(jnp.float32)
        fk = jax.lax.dot_general(
            oh, pts, (((1,), (1,)), ((), ())),
            preferred_element_type=jnp.float32)
        feats.append(fk)
    f = jnp.stack(feats, axis=1)

    rel = f[:, :, :3] - qc[:, None, :]
    dens = f[:, :, 3:4]
    fch = jnp.broadcast_to(qf.astype(jnp.float32)[:, None, None],
                           (nb, _K, 1))
    o_ref[0] = jnp.concatenate([rel, dens, fch], axis=-1)


def kernel(sample_points, points_frames, K):
    del K
    b, n, _ = sample_points.shape
    _, f, p, c = points_frames.shape
    fp = f * p
    pts_t = points_frames.reshape(b, fp, c).transpose(0, 2, 1)

    grid = (b, n // _NB)
    return pl.pallas_call(
        _knn_body,
        grid=grid,
        in_specs=[
            pl.BlockSpec((1, _NB, 5), lambda bi, ni: (bi, ni, 0)),
            pl.BlockSpec((1, c, fp), lambda bi, ni: (bi, 0, 0)),
        ],
        out_specs=pl.BlockSpec((1, _NB, _K, 5),
                               lambda bi, ni: (bi, ni, 0, 0)),
        out_shape=jax.ShapeDtypeStruct((b, n, _K, 5), jnp.float32),
    )(sample_points, pts_t)


# speedup vs baseline: 796.1609x; 1.4010x over previous
import jax
import jax.numpy as jnp
from jax import lax
from jax.experimental import pallas as pl
from jax.experimental.pallas import tpu as pltpu
from jax.experimental.pallas import tpu_sc as plsc

_K = 16
_L = 16
_NW = 32
_QPW = 32


def _sc_body(pts_hbm, q_hbm, dens_hbm, out_hbm, pts_v, q_v, ob_v, sem):
    fp = pts_hbm.shape[1] // 3
    p = fp // 4
    steps = p // _L
    wid = lax.axis_index("s") * 2 + lax.axis_index("c")
    b = wid // (_NW // pts_hbm.shape[0])

    pltpu.sync_copy(pts_hbm.at[b], pts_v)
    pltpu.sync_copy(q_hbm.at[wid], q_v)

    lanes = lax.iota(jnp.int32, _L)
    inf = jnp.float32(jnp.inf)

    def _splat(slot):
        return plsc.load_gather(q_v, [jnp.full((_L,), slot, jnp.int32)])

    def per_query(i, carry):
        qx = _splat(i)
        qy = _splat(_QPW + i)
        qz = _splat(2 * _QPW + i)
        qff = _splat(4 * _QPW + i)
        qfi = qff.astype(jnp.int32) - 1
        basev = qfi * p + lanes

        def scan_step(j, tk):
            td, ti, thr = tk
            idxv = basev + j * _L
            xv = plsc.load_gather(pts_v, [idxv])
            yv = plsc.load_gather(pts_v, [idxv + fp])
            zv = plsc.load_gather(pts_v, [idxv + 2 * fp])
            dx = xv - qx
            dy = yv - qy
            dz = zv - qz
            d2 = dx * dx + dy * dy + dz * dz

            def merge(td, ti, thr):
                cd, ci = plsc.sort_key_val(d2, idxv)
                rd = lax.rev(td, (0,))
                ri = lax.rev(ti, (0,))
                take = (cd < rd) | ((cd == rd) & (ci < ri))
                nd = jnp.where(take, cd, rd)
                ni = jnp.where(take, ci, ri)
                nd, ni = plsc.sort_key_val(nd, ni)
                return nd, ni, jnp.broadcast_to(nd[_L - 1], (_L,))

            def keep(td, ti, thr):
                return td, ti, thr

            return lax.cond(jnp.any(d2 < thr), merge, keep, td, ti, thr)

        td0 = jnp.full((_L,), inf, jnp.float32)
        ti0 = jnp.zeros((_L,), jnp.int32)
        td, ti, _ = lax.fori_loop(0, steps, scan_step, (td0, ti0, td0))

        xg = plsc.load_gather(pts_v, [ti])
        yg = plsc.load_gather(pts_v, [ti + fp])
        zg = plsc.load_gather(pts_v, [ti + 2 * fp])
        row = i * (5 * _K)
        ob_v[pl.ds(row, _K)] = xg - qx
        ob_v[pl.ds(row + _K, _K)] = yg - qy
        ob_v[pl.ds(row + 2 * _K, _K)] = zg - qz
        ob_v[pl.ds(row + 4 * _K, _K)] = qff - 1.0
        didx = ti + b * fp
        pltpu.async_copy(dens_hbm.at[didx],
                         ob_v.at[pl.ds(row + 3 * _K, _K)], sem).wait()
        return carry

    lax.fori_loop(0, _QPW, per_query, 0)
    pltpu.sync_copy(ob_v, out_hbm.at[pl.ds(wid * (_QPW * 5 * _K),
                                           _QPW * 5 * _K)])


def kernel(sample_points, points_frames, K):
    del K
    b, n, _ = sample_points.shape
    _, f, p, c = points_frames.shape
    fp = f * p
    pts = points_frames.reshape(b, fp, c)
    pts_xyz = pts[..., :3].transpose(0, 2, 1).reshape(b, 3 * fp)
    dens = pts[..., 3].reshape(b * fp)
    q = sample_points.reshape(b * n, 5).T
    qg = (q.reshape(5, _NW, _QPW).transpose(1, 0, 2)
          .reshape(_NW, 5 * _QPW))

    out = pl.kernel(
        _sc_body,
        out_type=jax.ShapeDtypeStruct((b * n * 5 * _K,), jnp.float32),
        mesh=plsc.VectorSubcoreMesh(core_axis_name="c", subcore_axis_name="s"),
        compiler_params=pltpu.CompilerParams(needs_layout_passes=False),
        scratch_types=[
            pltpu.VMEM((3 * fp,), jnp.float32),
            pltpu.VMEM((5 * _QPW,), jnp.float32),
            pltpu.VMEM((_QPW * 5 * _K,), jnp.float32),
            pltpu.SemaphoreType.DMA,
        ],
    )(pts_xyz, qg, dens)
    return out.reshape(b, n, 5, _K).transpose(0, 1, 3, 2)
